# baseline (speedup 1.0000x reference)
import functools

import jax
import jax.numpy as jnp
from jax import lax
from jax.experimental import pallas as pl
from jax.experimental.pallas import tpu as pltpu
from jax.experimental.pallas import tpu_sc as plsc

B, L = 4096, 50
N = B * L
EMBED = 64
GENRE_ROWS = 1000

NC, NS = 2, 16
NW = NC * NS
ROWS_W = B // NW
PER_W = ROWS_W * L
NBUF = 8
N_GROUPS = PER_W // 16
N_STEPS = N_GROUPS // NBUF


def _advance(row, col):
    col = col + 16
    wrap = (col >= L).astype(jnp.int32)
    return row + wrap, col - wrap * L


def _sc_body(uid_hbm, gid_hbm, utab_hbm, gtab_hbm, out_hbm,
             gtab_v, uidx_v, gidx_v, out_v, bufs_v, *sems):
    wid = lax.axis_index("c") * NS + lax.axis_index("s")
    b0 = wid * ROWS_W

    gsem = sems[NBUF]
    cpu = pltpu.async_copy(uid_hbm.at[pl.ds(b0, ROWS_W)], uidx_v, sems[0])
    cpg = pltpu.async_copy(gid_hbm.at[pl.ds(b0, ROWS_W)], gidx_v, sems[1])
    cpt = pltpu.async_copy(gtab_hbm, gtab_v, gsem)
    cpu.wait()
    cpg.wait()
    cpt.wait()

    lane = lax.iota(jnp.int32, 16)
    row0 = jnp.zeros((16,), jnp.int32)

    def fire(g_row, g_col, slot):
        idx = plsc.load_gather(uidx_v, [g_row, g_col])
        pltpu.async_copy(utab_hbm.at[idx], bufs_v.at[slot], sems[slot])

    frow, fcol = row0, lane
    for k in range(NBUF):
        fire(frow, fcol, k)
        frow, fcol = _advance(frow, fcol)

    def step(s, carry):
        frow, fcol, crow, ccol, col0 = carry
        for k in range(NBUF):
            pltpu.make_async_copy(utab_hbm.at[lane], bufs_v.at[k],
                                  sems[k]).wait()
            gidx = plsc.load_gather(gidx_v, [crow, ccol])
            acc = jnp.zeros((16,), jnp.float32)
            col = col0
            buf2d = bufs_v.at[k]
            for _ in range(EMBED):
                u = plsc.load_gather(buf2d, [lane, col])
                gv = plsc.load_gather(gtab_v, [gidx, col])
                acc = acc + u * gv
                col = (col + 1) & (EMBED - 1)
            plsc.store_scatter(out_v, [crow, ccol], acc)
            fire(frow, fcol, k)
            frow, fcol = _advance(frow, fcol)
            crow, ccol = _advance(crow, ccol)
        return frow, fcol, crow, ccol, col0

    carry = (frow, fcol, row0, lane, lane)
    lax.fori_loop(0, N_STEPS, step, carry)

    for k in range(NBUF):
        pltpu.make_async_copy(utab_hbm.at[lane], bufs_v.at[k], sems[k]).wait()

    pltpu.sync_copy(out_v, out_hbm.at[pl.ds(b0, ROWS_W)])


@jax.jit
def _sc_ranker(uid, gid, utab, gtab):
    mesh = plsc.VectorSubcoreMesh(core_axis_name="c", subcore_axis_name="s")
    fn = functools.partial(
        pl.kernel,
        out_type=jax.ShapeDtypeStruct((B, L), jnp.float32),
        mesh=mesh,
        scratch_types=[
            pltpu.VMEM((GENRE_ROWS, EMBED), jnp.float32),
            pltpu.VMEM((ROWS_W, L), jnp.int32),
            pltpu.VMEM((ROWS_W, L), jnp.int32),
            pltpu.VMEM((ROWS_W, L), jnp.float32),
            pltpu.VMEM((NBUF, 16, EMBED), jnp.float32),
        ] + [pltpu.SemaphoreType.DMA] * (NBUF + 1),
        compiler_params=pltpu.CompilerParams(needs_layout_passes=False,
                                             use_tc_tiling_on_sc=False,
                                             disable_bounds_checks=True,
                                             disable_semaphore_checks=True),
    )(_sc_body)
    return fn(uid, gid, utab, gtab)


def kernel(user_id, title, genre, user_table, title_table, genre_table):
    return _sc_ranker(user_id, genre, user_table, genre_table)
